# baseline (speedup 1.0000x reference)
import jax
import jax.numpy as jnp
from jax.experimental import pallas as pl
from jax.experimental.pallas import tpu as pltpu

N = 4096
NH = N // 2
HID = 64
O = 128
THRESH = 0.6
ABLK = 256
SBLK = 256
CBLK = 512
NA = N // ABLK
NB = N // SBLK
NC = N // CBLK


def _u_kernel(f1_ref, f2_ref, w11_ref, w21_ref, u1_ref, u2_ref):
    u1_ref[...] = jnp.dot(f1_ref[...], w11_ref[...],
                          preferred_element_type=jnp.float32)
    u2_ref[...] = jnp.dot(f2_ref[...], w21_ref[...],
                          preferred_element_type=jnp.float32)


def _u(feat1, feat2, e1w1, e2w1):
    d1 = feat1.shape[1]
    d2 = feat2.shape[1]
    blk = 512
    return pl.pallas_call(
        _u_kernel,
        grid=(N // blk,),
        in_specs=[
            pl.BlockSpec((blk, d1), lambda i: (i, 0)),
            pl.BlockSpec((blk, d2), lambda i: (i, 0)),
            pl.BlockSpec((d1, HID), lambda i: (0, 0)),
            pl.BlockSpec((d2, HID), lambda i: (0, 0)),
        ],
        out_specs=[
            pl.BlockSpec((blk, HID), lambda i: (i, 0)),
            pl.BlockSpec((blk, HID), lambda i: (i, 0)),
        ],
        out_shape=[
            jax.ShapeDtypeStruct((N, HID), jnp.float32),
            jax.ShapeDtypeStruct((N, HID), jnp.float32),
        ],
    )(feat1, feat2, e1w1, e2w1)


def _chain_body(a_ref, u_ref, w2_ref, dw_ref, yprev_ref,
                y_ref, recon_ref, z_ref,
                adjbf_ref, hn_ref, yin_ref, x_ref):
    i = pl.program_id(0)

    @pl.when(i < NA)
    def _phase_a():
        a = a_ref[...]
        h = jnp.dot(a, u_ref[...], preferred_element_type=jnp.float32)
        h = jnp.maximum(h, 0.0)
        norm = jnp.sqrt(jnp.sum(h * h, axis=1, keepdims=True))
        hn = h / jnp.maximum(norm, 1e-12)
        hn_ref[pl.ds(i * ABLK, ABLK), :] = hn.astype(jnp.bfloat16)
        yin_ref[pl.ds(i * ABLK, ABLK), :] = jnp.dot(
            h, w2_ref[...], preferred_element_type=jnp.float32
        ).astype(jnp.bfloat16)
        adjbf_ref[pl.ds(i * ABLK, ABLK), :] = a.astype(jnp.bfloat16)

    @pl.when(jnp.logical_and(i >= NA, i < NA + NB))
    def _phase_b():
        j = i - NA
        hnb = hn_ref[pl.ds(j * SBLK, SBLK), :]
        s = jax.lax.dot_general(
            hnb, hn_ref[...],
            dimension_numbers=(((1,), (1,)), ((), ())),
            preferred_element_type=jnp.float32)
        s = jnp.where(s < THRESH, 0.0, s)
        rs = jnp.sum(s, axis=1, keepdims=True)
        agg = jnp.dot(s.astype(jnp.bfloat16), yin_ref[...],
                      preferred_element_type=jnp.float32)
        y = agg / jnp.maximum(rs, 1e-12)
        y_ref[...] = y
        x_ref[pl.ds(j * SBLK, SBLK), :] = jnp.dot(
            y, dw_ref[...], preferred_element_type=jnp.float32
        ).astype(jnp.bfloat16)
        if z_ref is not None:
            z_ref[...] = (y + yprev_ref[...]) * 0.5

    @pl.when(i >= NA + NB)
    def _phase_c():
        k = i - (NA + NB)
        recon_ref[...] = jnp.dot(
            adjbf_ref[pl.ds(k * CBLK, CBLK), :], x_ref[...],
            preferred_element_type=jnp.float32)


def _chain1_kernel(a_ref, u_ref, w2_ref, dw_ref,
                   y_ref, recon_ref,
                   adjbf_ref, hn_ref, yin_ref, x_ref):
    _chain_body(a_ref, u_ref, w2_ref, dw_ref, None,
                y_ref, recon_ref, None,
                adjbf_ref, hn_ref, yin_ref, x_ref)


def _chain2_kernel(a_ref, u_ref, w2_ref, dw_ref, yprev_ref,
                   y_ref, recon_ref, z_ref,
                   adjbf_ref, hn_ref, yin_ref, x_ref):
    _chain_body(a_ref, u_ref, w2_ref, dw_ref, yprev_ref,
                y_ref, recon_ref, z_ref,
                adjbf_ref, hn_ref, yin_ref, x_ref)


def _chain(adj, u, w2, dw, yprev=None):
    d = dw.shape[1]
    grid = (NA + NB + NC,)
    in_specs = [
        pl.BlockSpec((ABLK, N), lambda i: (jnp.minimum(i, NA - 1), 0)),
        pl.BlockSpec((N, HID), lambda i: (0, 0)),
        pl.BlockSpec((HID, O), lambda i: (0, 0)),
        pl.BlockSpec((O, d), lambda i: (0, 0)),
    ]
    out_specs = [
        pl.BlockSpec((SBLK, O),
                     lambda i: (jnp.clip(i - NA, 0, NB - 1), 0)),
        pl.BlockSpec((CBLK, d),
                     lambda i: (jnp.clip(i - NA - NB, 0, NC - 1), 0)),
    ]
    out_shape = [
        jax.ShapeDtypeStruct((N, O), jnp.float32),
        jax.ShapeDtypeStruct((N, d), jnp.float32),
    ]
    scratch_shapes = [
        pltpu.VMEM((N, N), jnp.bfloat16),
        pltpu.VMEM((N, HID), jnp.bfloat16),
        pltpu.VMEM((N, O), jnp.bfloat16),
        pltpu.VMEM((N, d), jnp.bfloat16),
    ]
    args = [adj, u, w2, dw]
    body = _chain1_kernel
    if yprev is not None:
        in_specs.append(
            pl.BlockSpec((SBLK, O),
                         lambda i: (jnp.clip(i - NA, 0, NB - 1), 0)))
        out_specs.append(
            pl.BlockSpec((SBLK, O),
                         lambda i: (jnp.clip(i - NA, 0, NB - 1), 0)))
        out_shape.append(jax.ShapeDtypeStruct((N, O), jnp.float32))
        args.append(yprev)
        body = _chain2_kernel
    return pl.pallas_call(
        body,
        grid=grid,
        in_specs=in_specs,
        out_specs=out_specs,
        out_shape=out_shape,
        scratch_shapes=scratch_shapes,
    )(*args)


def kernel(feat1, feat2, adj_spatial1, adj_spatial2,
           e1w1, e1w2, e2w1, e2w2, d1w, d2w):
    u1, u2 = _u(feat1, feat2, e1w1, e2w1)
    y1, recon1 = _chain(adj_spatial1, u1, e1w2, d1w)
    y2, recon2, z = _chain(adj_spatial2, u2, e2w2, d2w, y1)
    return (y1, y2, z, recon1, recon2)


# speedup vs baseline: 1.0630x; 1.0630x over previous
import jax
import jax.numpy as jnp
from jax.experimental import pallas as pl
from jax.experimental.pallas import tpu as pltpu

N = 4096
NH = N // 2
HID = 64
O = 128
THRESH = 0.6
ABLK = 256
SBLK = 256
CBLK = 512
NA = N // ABLK
NB = N // SBLK
NC = N // CBLK


def _u_kernel(f1_ref, f2_ref, w11_ref, w21_ref, u1_ref, u2_ref):
    u1_ref[...] = jnp.dot(f1_ref[...], w11_ref[...],
                          preferred_element_type=jnp.float32)
    u2_ref[...] = jnp.dot(f2_ref[...], w21_ref[...],
                          preferred_element_type=jnp.float32)


def _u(feat1, feat2, e1w1, e2w1):
    d1 = feat1.shape[1]
    d2 = feat2.shape[1]
    blk = 512
    return pl.pallas_call(
        _u_kernel,
        grid=(N // blk,),
        in_specs=[
            pl.BlockSpec((blk, d1), lambda i: (i, 0)),
            pl.BlockSpec((blk, d2), lambda i: (i, 0)),
            pl.BlockSpec((d1, HID), lambda i: (0, 0)),
            pl.BlockSpec((d2, HID), lambda i: (0, 0)),
        ],
        out_specs=[
            pl.BlockSpec((blk, HID), lambda i: (i, 0)),
            pl.BlockSpec((blk, HID), lambda i: (i, 0)),
        ],
        out_shape=[
            jax.ShapeDtypeStruct((N, HID), jnp.float32),
            jax.ShapeDtypeStruct((N, HID), jnp.float32),
        ],
    )(feat1, feat2, e1w1, e2w1)


def _chain_body(a_ref, u_ref, w2_ref, dw_ref, yprev_ref,
                y_ref, recon_ref, z_ref,
                adjbf_ref, hn_ref, yin_ref, x_ref):
    i = pl.program_id(0)

    @pl.when(i < NA)
    def _phase_a():
        a = a_ref[...]
        h = jnp.dot(a, u_ref[...], preferred_element_type=jnp.float32)
        h = jnp.maximum(h, 0.0)
        norm = jnp.sqrt(jnp.sum(h * h, axis=1, keepdims=True))
        hn = h / jnp.maximum(norm, 1e-12)
        hn_ref[pl.ds(i * ABLK, ABLK), :] = hn
        yin_ref[pl.ds(i * ABLK, ABLK), :] = jnp.dot(
            h, w2_ref[...], preferred_element_type=jnp.float32)
        adjbf_ref[pl.ds(i * ABLK, ABLK), :] = a.astype(jnp.bfloat16)

    @pl.when(jnp.logical_and(i >= NA, i < NA + NB))
    def _phase_b():
        j = i - NA
        hnb = hn_ref[pl.ds(j * SBLK, SBLK), :]
        s = jax.lax.dot_general(
            hnb, hn_ref[...],
            dimension_numbers=(((1,), (1,)), ((), ())),
            preferred_element_type=jnp.float32)
        s = jnp.where(s < THRESH, 0.0, s)
        rs = jnp.sum(s, axis=1, keepdims=True)
        agg = jnp.dot(s, yin_ref[...], preferred_element_type=jnp.float32)
        y = agg / jnp.maximum(rs, 1e-12)
        y_ref[...] = y
        x_ref[pl.ds(j * SBLK, SBLK), :] = jnp.dot(
            y, dw_ref[...], preferred_element_type=jnp.float32
        ).astype(jnp.bfloat16)
        if z_ref is not None:
            z_ref[...] = (y + yprev_ref[...]) * 0.5

    @pl.when(i >= NA + NB)
    def _phase_c():
        k = i - (NA + NB)
        recon_ref[...] = jnp.dot(
            adjbf_ref[pl.ds(k * CBLK, CBLK), :], x_ref[...],
            preferred_element_type=jnp.float32)


def _chain1_kernel(a_ref, u_ref, w2_ref, dw_ref,
                   y_ref, recon_ref,
                   adjbf_ref, hn_ref, yin_ref, x_ref):
    _chain_body(a_ref, u_ref, w2_ref, dw_ref, None,
                y_ref, recon_ref, None,
                adjbf_ref, hn_ref, yin_ref, x_ref)


def _chain2_kernel(a_ref, u_ref, w2_ref, dw_ref, yprev_ref,
                   y_ref, recon_ref, z_ref,
                   adjbf_ref, hn_ref, yin_ref, x_ref):
    _chain_body(a_ref, u_ref, w2_ref, dw_ref, yprev_ref,
                y_ref, recon_ref, z_ref,
                adjbf_ref, hn_ref, yin_ref, x_ref)


def _chain(adj, u, w2, dw, yprev=None):
    d = dw.shape[1]
    grid = (NA + NB + NC,)
    in_specs = [
        pl.BlockSpec((ABLK, N), lambda i: (jnp.minimum(i, NA - 1), 0)),
        pl.BlockSpec((N, HID), lambda i: (0, 0)),
        pl.BlockSpec((HID, O), lambda i: (0, 0)),
        pl.BlockSpec((O, d), lambda i: (0, 0)),
    ]
    out_specs = [
        pl.BlockSpec((SBLK, O),
                     lambda i: (jnp.clip(i - NA, 0, NB - 1), 0)),
        pl.BlockSpec((CBLK, d),
                     lambda i: (jnp.clip(i - NA - NB, 0, NC - 1), 0)),
    ]
    out_shape = [
        jax.ShapeDtypeStruct((N, O), jnp.float32),
        jax.ShapeDtypeStruct((N, d), jnp.float32),
    ]
    scratch_shapes = [
        pltpu.VMEM((N, N), jnp.bfloat16),
        pltpu.VMEM((N, HID), jnp.float32),
        pltpu.VMEM((N, O), jnp.float32),
        pltpu.VMEM((N, d), jnp.bfloat16),
    ]
    args = [adj, u, w2, dw]
    body = _chain1_kernel
    if yprev is not None:
        in_specs.append(
            pl.BlockSpec((SBLK, O),
                         lambda i: (jnp.clip(i - NA, 0, NB - 1), 0)))
        out_specs.append(
            pl.BlockSpec((SBLK, O),
                         lambda i: (jnp.clip(i - NA, 0, NB - 1), 0)))
        out_shape.append(jax.ShapeDtypeStruct((N, O), jnp.float32))
        args.append(yprev)
        body = _chain2_kernel
    return pl.pallas_call(
        body,
        grid=grid,
        in_specs=in_specs,
        out_specs=out_specs,
        out_shape=out_shape,
        scratch_shapes=scratch_shapes,
    )(*args)


def kernel(feat1, feat2, adj_spatial1, adj_spatial2,
           e1w1, e1w2, e2w1, e2w2, d1w, d2w):
    u1, u2 = _u(feat1, feat2, e1w1, e2w1)
    y1, recon1 = _chain(adj_spatial1, u1, e1w2, d1w)
    y2, recon2, z = _chain(adj_spatial2, u2, e2w2, d2w, y1)
    return (y1, y2, z, recon1, recon2)
